# baseline (speedup 1.0000x reference)
import jax
import jax.numpy as jnp
from jax.experimental import pallas as pl
from jax.experimental.pallas import tpu as pltpu

_OUT_DOM = 64
_DIM_CONT = 128


def _body(ids_ref, dvec_ref, w_ref, b_ref, x_hbm, t_hbm, out_hbm,
          rowbuf, dvbuf, xsem, rsem, dsem):
    bsz = x_hbm.shape[0]
    cin = x_hbm.shape[1]
    hw = x_hbm.shape[2]
    s = dvbuf.shape[2]
    ns = hw // s

    xcps = []
    for i in range(bsz):
        cp = pltpu.make_async_copy(
            x_hbm.at[i], out_hbm.at[i, pl.ds(0, cin)], xsem.at[i])
        cp.start()
        xcps.append(cp)

    rcps = []
    for i in range(bsz):
        idx = ids_ref[i]
        cp = pltpu.make_async_copy(
            t_hbm.at[pl.ds(idx, 1)], rowbuf.at[pl.ds(i, 1)], rsem.at[i])
        cp.start()
        rcps.append(cp)
    for cp in rcps:
        cp.wait()

    dv = jnp.maximum(
        jnp.dot(dvec_ref[...], w_ref[...], preferred_element_type=jnp.float32)
        + b_ref[...],
        0.0,
    ) + rowbuf[...]
    dvbuf[...] = jnp.broadcast_to(dv[:, :, None], (bsz, _OUT_DOM, s))

    dcps = []
    for i in range(bsz):
        for j in range(ns):
            cp = pltpu.make_async_copy(
                dvbuf.at[i],
                out_hbm.at[i, pl.ds(cin, _OUT_DOM), pl.ds(j * s, s)],
                dsem.at[i, j])
            cp.start()
            dcps.append(cp)
    for cp in dcps:
        cp.wait()
    for cp in xcps:
        cp.wait()


def kernel(x, domain_ids, domain_vectors, W, b, table):
    bsz, cin, h, w = x.shape
    hw = h * w
    cout = cin + _OUT_DOM
    ns = 8
    s = hw // ns

    x3 = x.reshape(bsz, cin, hw)
    b2 = b.reshape(1, _OUT_DOM)

    out = pl.pallas_call(
        _body,
        in_specs=[
            pl.BlockSpec(memory_space=pltpu.SMEM),
            pl.BlockSpec(memory_space=pltpu.VMEM),
            pl.BlockSpec(memory_space=pltpu.VMEM),
            pl.BlockSpec(memory_space=pltpu.VMEM),
            pl.BlockSpec(memory_space=pltpu.HBM),
            pl.BlockSpec(memory_space=pltpu.HBM),
        ],
        out_specs=pl.BlockSpec(memory_space=pltpu.HBM),
        out_shape=jax.ShapeDtypeStruct((bsz, cout, hw), x.dtype),
        scratch_shapes=[
            pltpu.VMEM((bsz, _OUT_DOM), jnp.float32),
            pltpu.VMEM((bsz, _OUT_DOM, s), jnp.float32),
            pltpu.SemaphoreType.DMA((bsz,)),
            pltpu.SemaphoreType.DMA((bsz,)),
            pltpu.SemaphoreType.DMA((bsz, ns)),
        ],
    )(domain_ids, domain_vectors, W, b2, x3, table)
    return out.reshape(bsz, cout, h, w)


# speedup vs baseline: 7.6936x; 7.6936x over previous
import jax
import jax.numpy as jnp
from jax.experimental import pallas as pl
from jax.experimental.pallas import tpu as pltpu

_OUT_DOM = 64
_DIM_CONT = 128


def _body(ids_ref, x_ref, dvec_ref, w_ref, b_ref, trow_ref, out_ref):
    cin = x_ref.shape[1]
    s = x_ref.shape[2]
    out_ref[0, :cin, :] = x_ref[0]
    dvv = dvec_ref[0]
    dv = jnp.maximum(
        jnp.dot(dvv, w_ref[...], preferred_element_type=jnp.float32) + b_ref[...],
        0.0,
    )
    dv = dv + trow_ref[0]
    out_ref[0, cin:, :] = jnp.broadcast_to(dv.reshape(_OUT_DOM, 1), (_OUT_DOM, s))


def kernel(x, domain_ids, domain_vectors, W, b, table):
    bsz, cin, h, w = x.shape
    hw = h * w
    cout = cin + _OUT_DOM
    ns = 8
    s = hw // ns

    x3 = x.reshape(bsz, cin, hw)
    t3 = table.reshape(table.shape[0], 1, _OUT_DOM)
    dvec3 = domain_vectors.reshape(bsz, 1, _DIM_CONT)
    b2 = b.reshape(1, _OUT_DOM)

    out = pl.pallas_call(
        _body,
        grid_spec=pltpu.PrefetchScalarGridSpec(
            num_scalar_prefetch=1,
            grid=(bsz, ns),
            in_specs=[
                pl.BlockSpec((1, cin, s), lambda i, j, ids: (i, 0, j)),
                pl.BlockSpec((1, 1, _DIM_CONT), lambda i, j, ids: (i, 0, 0)),
                pl.BlockSpec((_DIM_CONT, _OUT_DOM), lambda i, j, ids: (0, 0)),
                pl.BlockSpec((1, _OUT_DOM), lambda i, j, ids: (0, 0)),
                pl.BlockSpec((1, 1, _OUT_DOM), lambda i, j, ids: (ids[i], 0, 0)),
            ],
            out_specs=pl.BlockSpec((1, cout, s), lambda i, j, ids: (i, 0, j)),
        ),
        out_shape=jax.ShapeDtypeStruct((bsz, cout, hw), x.dtype),
    )(domain_ids, x3, dvec3, W, b2, t3)
    return out.reshape(bsz, cout, h, w)


# speedup vs baseline: 14.2680x; 1.8545x over previous
import jax
import jax.numpy as jnp
from jax.experimental import pallas as pl
from jax.experimental.pallas import tpu as pltpu

_OUT_DOM = 64
_DIM_CONT = 128


def _body(ids_ref, x_ref, dvec_ref, w_ref, b_ref, trow_ref, out_ref):
    cin = x_ref.shape[1]
    hc = x_ref.shape[2]
    wd = x_ref.shape[3]
    out_ref[0, :cin] = x_ref[0]
    dvv = dvec_ref[0]
    dv = jnp.maximum(
        jnp.dot(dvv, w_ref[...], preferred_element_type=jnp.float32) + b_ref[...],
        0.0,
    )
    dv = dv + trow_ref[0]
    out_ref[0, cin:] = jnp.broadcast_to(
        dv.reshape(_OUT_DOM, 1, 1), (_OUT_DOM, hc, wd))


def kernel(x, domain_ids, domain_vectors, W, b, table):
    bsz, cin, h, w = x.shape
    cout = cin + _OUT_DOM
    hc = 32
    nh = h // hc

    t3 = table.reshape(table.shape[0], 1, _OUT_DOM)
    dvec3 = domain_vectors.reshape(bsz, 1, _DIM_CONT)
    b2 = b.reshape(1, _OUT_DOM)

    return pl.pallas_call(
        _body,
        grid_spec=pltpu.PrefetchScalarGridSpec(
            num_scalar_prefetch=1,
            grid=(bsz, nh),
            in_specs=[
                pl.BlockSpec((1, cin, hc, w), lambda i, j, ids: (i, 0, j, 0)),
                pl.BlockSpec((1, 1, _DIM_CONT), lambda i, j, ids: (i, 0, 0)),
                pl.BlockSpec((_DIM_CONT, _OUT_DOM), lambda i, j, ids: (0, 0)),
                pl.BlockSpec((1, _OUT_DOM), lambda i, j, ids: (0, 0)),
                pl.BlockSpec((1, 1, _OUT_DOM), lambda i, j, ids: (ids[i], 0, 0)),
            ],
            out_specs=pl.BlockSpec((1, cout, hc, w), lambda i, j, ids: (i, 0, j, 0)),
        ),
        out_shape=jax.ShapeDtypeStruct((bsz, cout, h, w), x.dtype),
    )(domain_ids, x, dvec3, W, b2, t3)
